# baseline (speedup 1.0000x reference)
import jax
import jax.numpy as jnp
from jax import lax
from jax.experimental import pallas as pl
from jax.experimental.pallas import tpu as pltpu
from jax.experimental.pallas import tpu_sc as plsc

_NUM_FIELDS = 26
_VOCAB = 100000
_LOW = 49920
_HIGH = _VOCAB - _LOW
_EMBED_DIM = 32
_BATCH = 16384
_NC, _NS, _L = 2, 16, 16
_NW = _NC * _NS
_CHUNK = 2048
_NCH = _BATCH // _CHUNK
_GRP = _CHUNK // _L
_UNROLL = 8


def _body(tok_hbm, tab_hbm, out_hbm, tokc, row0, row1, outv,
          rsem0, rsem1, tsem0, tsem1, ssem):
    d = lax.axis_index("s") * _NC + lax.axis_index("c")
    lanes = lax.iota(jnp.int32, _L)
    tsems = (tsem0, tsem1)

    def tok_issue(i, c, slot):
        pltpu.async_copy(
            tok_hbm.at[i, pl.ds(c * _CHUNK, _CHUNK)], tokc.at[slot],
            tsems[slot])

    def tok_wait(slot):
        pltpu.make_async_copy(
            tok_hbm.at[0, pl.ds(0, _CHUNK)], tokc.at[slot],
            tsems[slot]).wait()

    def do_pass(i, which):
        tok_issue(i, 0, 0)
        for c in range(_NCH):
            slot = c & 1
            if c + 1 < _NCH:
                tok_issue(i, c + 1, 1 - slot)
            tok_wait(slot)

            def grp(g8, _, c=c, slot=slot):
                for k in range(_UNROLL):
                    off = g8 * (_UNROLL * _L) + k * _L
                    idx = tokc[slot, pl.ds(off, _L)]
                    gof = c * _CHUNK + off
                    if which == 0:
                        v = plsc.load_gather(
                            row0, [jnp.minimum(idx, _LOW - 1)])
                        outv[pl.ds(gof, _L)] = v
                    else:
                        m = idx >= _LOW
                        idx2 = jnp.maximum(idx - _LOW, 0)
                        v = plsc.load_gather(row1, [idx2], mask=m)
                        plsc.store_scatter(outv, [lanes + gof], v, mask=m)
                return _

            lax.fori_loop(0, _GRP // _UNROLL, grp, 0)
            if which == 1:
                r = i * _EMBED_DIM + d
                pltpu.async_copy(
                    outv.at[pl.ds(c * _CHUNK, _CHUNK)],
                    out_hbm.at[r, pl.ds(c * _CHUNK, _CHUNK)], ssem)

    pltpu.async_copy(tab_hbm.at[0, d, pl.ds(0, _LOW)], row0, rsem0)

    def task(i, carry):
        pltpu.async_copy(tab_hbm.at[i, d, pl.ds(_LOW, _HIGH)], row1, rsem1)

        @pl.when(i > 0)
        def _():
            for _c in range(_NCH):
                pltpu.make_async_copy(
                    outv.at[pl.ds(0, _CHUNK)],
                    out_hbm.at[0, pl.ds(0, _CHUNK)], ssem).wait()

        pltpu.make_async_copy(
            tab_hbm.at[0, 0, pl.ds(0, _LOW)], row0, rsem0).wait()
        do_pass(i, 0)

        @pl.when(i + 1 < _NUM_FIELDS)
        def _():
            pltpu.async_copy(
                tab_hbm.at[i + 1, d, pl.ds(0, _LOW)], row0, rsem0)

        pltpu.make_async_copy(
            tab_hbm.at[0, 0, pl.ds(_LOW, _HIGH)], row1, rsem1).wait()
        do_pass(i, 1)
        return carry

    lax.fori_loop(0, _NUM_FIELDS, task, 0)
    for _c in range(_NCH):
        pltpu.make_async_copy(
            outv.at[pl.ds(0, _CHUNK)],
            out_hbm.at[0, pl.ds(0, _CHUNK)], ssem).wait()


def kernel(tokens, tables):
    tok = tokens.T.astype(jnp.int32)
    tab = tables.transpose(0, 2, 1)
    mesh = plsc.VectorSubcoreMesh(core_axis_name="c", subcore_axis_name="s")
    run = pl.kernel(
        _body,
        mesh=mesh,
        out_type=jax.ShapeDtypeStruct(
            (_NUM_FIELDS * _EMBED_DIM, _BATCH), jnp.float32),
        scratch_types=[
            pltpu.VMEM((2, _CHUNK), jnp.int32),
            pltpu.VMEM((_LOW,), jnp.float32),
            pltpu.VMEM((_HIGH,), jnp.float32),
            pltpu.VMEM((_BATCH,), jnp.float32),
            pltpu.SemaphoreType.DMA,
            pltpu.SemaphoreType.DMA,
            pltpu.SemaphoreType.DMA,
            pltpu.SemaphoreType.DMA,
            pltpu.SemaphoreType.DMA,
        ],
        compiler_params=pltpu.CompilerParams(
            use_tc_tiling_on_sc=True, needs_layout_passes=False),
    )
    out_t = run(tok, tab)
    return out_t.T


# speedup vs baseline: 1.6192x; 1.6192x over previous
import jax
import jax.numpy as jnp
from jax import lax
from jax.experimental import pallas as pl
from jax.experimental.pallas import tpu as pltpu
from jax.experimental.pallas import tpu_sc as plsc

_NUM_FIELDS = 26
_VOCAB = 100000
_EMBED_DIM = 32
_BATCH = 16384
_NC, _NS, _L = 2, 16, 16
_NW = _NC * _NS
_CHUNK = 2048
_NCH = _BATCH // _CHUNK
_GRP = _CHUNK // _L


def _body(tok_hbm, tab_hbm, out_hbm, tokv, rowv, outv, ssem0, ssem1, rsem):
    d = lax.axis_index("s") * _NC + lax.axis_index("c")
    ssems = (ssem0, ssem1)

    def task(i, carry):
        r = i * _EMBED_DIM + d
        rcp = pltpu.async_copy(tab_hbm.at[i, d], rowv, rsem)
        pltpu.sync_copy(tok_hbm.at[i], tokv)
        rcp.wait()
        for c in range(_NCH):
            slot = c & 1

            def wait_slot(slot=slot):
                pltpu.make_async_copy(
                    outv.at[slot], out_hbm.at[r, pl.ds(0, _CHUNK)],
                    ssems[slot]).wait()

            if c >= 2:
                wait_slot()
            else:
                pl.when(i > 0)(wait_slot)

            def grp(g8, _, c=c, slot=slot):
                for k in range(8):
                    off = g8 * (8 * _L) + k * _L
                    idx = tokv[c * 16 + g8, pl.ds(k * _L, _L)]
                    outv[slot, pl.ds(off, _L)] = plsc.load_gather(rowv, [idx])
                return _

            lax.fori_loop(0, _GRP // 8, grp, 0)
            pltpu.async_copy(
                outv.at[slot], out_hbm.at[r, pl.ds(c * _CHUNK, _CHUNK)],
                ssems[slot])
        return carry

    lax.fori_loop(0, _NUM_FIELDS, task, 0)
    for slot in range(2):
        pltpu.make_async_copy(
            outv.at[slot], out_hbm.at[0, pl.ds(0, _CHUNK)],
            ssems[slot]).wait()


def kernel(tokens, tables):
    tok = tokens.T.astype(jnp.int32).reshape(_NUM_FIELDS, 128, 128)
    tab = tables.transpose(0, 2, 1)
    mesh = plsc.VectorSubcoreMesh(core_axis_name="c", subcore_axis_name="s")
    run = pl.kernel(
        _body,
        mesh=mesh,
        out_type=jax.ShapeDtypeStruct(
            (_NUM_FIELDS * _EMBED_DIM, _BATCH), jnp.float32),
        scratch_types=[
            pltpu.VMEM((128, 128), jnp.int32),
            pltpu.VMEM((_VOCAB,), jnp.float32),
            pltpu.VMEM((2, _CHUNK), jnp.float32),
            pltpu.SemaphoreType.DMA,
            pltpu.SemaphoreType.DMA,
            pltpu.SemaphoreType.DMA,
        ],
        compiler_params=pltpu.CompilerParams(
            use_tc_tiling_on_sc=True, needs_layout_passes=False),
    )
    out_t = run(tok, tab)
    return out_t.T
